# Initial kernel scaffold; baseline (speedup 1.0000x reference)
import jax
import jax.numpy as jnp
from jax.experimental import pallas as pl


def kernel(inputs, token_table, pos_table):
    raise NotImplementedError("write your pallas kernel here")


# speedup vs baseline: 3.0084x; 3.0084x over previous
import functools

import jax
import jax.numpy as jnp
from jax import lax
from jax.experimental import pallas as pl
from jax.experimental.pallas import tpu as pltpu
from jax.experimental.pallas import tpu_sc as plsc

NUM_PATCHES = 1024
PATCH_DIM = 768
DIM_MODEL = 64
BATCH = 1024

CHUNK = 128
N_CHUNKS = BATCH * PATCH_DIM // CHUNK
NW = 32
CPW = N_CHUNKS // NW
CHUNKS_PER_ROW = PATCH_DIM // CHUNK
NBUF = 4
LANES = 16


def _body(idx_hbm, tok_hbm, pos_hbm, out_hbm, idx_v, pos_v,
          buf0, buf1, buf2, buf3, gs0, gs1, gs2, gs3, os0, os1, os2, os3):
    bufs = (buf0, buf1, buf2, buf3)
    gsems = (gs0, gs1, gs2, gs3)
    osems = (os0, os1, os2, os3)

    wid = lax.axis_index("s") * 2 + lax.axis_index("c")
    base = wid * CPW

    pltpu.sync_copy(pos_hbm, pos_v)
    pltpu.sync_copy(idx_hbm.at[pl.ds(base, CPW)], idx_v)

    def start_gather(b, cj):
        pltpu.async_copy(tok_hbm.at[idx_v.at[cj]], bufs[b], gsems[b])

    def wait_gather(b):
        pltpu.make_async_copy(tok_hbm.at[idx_v.at[0]], bufs[b], gsems[b]).wait()

    def start_out(b, cj):
        pltpu.async_copy(bufs[b], out_hbm.at[pl.ds((base + cj) * CHUNK, CHUNK)],
                         osems[b])

    def wait_out(b):
        pltpu.make_async_copy(bufs[b], out_hbm.at[pl.ds(0, CHUNK)],
                              osems[b]).wait()

    def add_pos(b, p0):
        buf = bufs[b]

        @pl.loop(0, CHUNK, unroll=4)
        def _(r):
            for c in range(DIM_MODEL // LANES):
                x = pos_v[p0 + r, pl.ds(c * LANES, LANES)]
                plsc.addupdate(buf.at[r, pl.ds(c * LANES, LANES)], x)

    for b in range(NBUF):
        start_gather(b, b)

    @pl.loop(0, CPW // NBUF - 1)
    def _(g):
        for b in range(NBUF):
            cj = g * NBUF + b
            wait_gather(b)
            add_pos(b, lax.rem(cj, CHUNKS_PER_ROW) * CHUNK)
            start_out(b, cj)
        for b in range(NBUF):
            wait_out(b)
            start_gather(b, g * NBUF + NBUF + b)

    for b in range(NBUF):
        cj = CPW - NBUF + b
        wait_gather(b)
        add_pos(b, (cj % CHUNKS_PER_ROW) * CHUNK)
        start_out(b, cj)
    for b in range(NBUF):
        wait_out(b)


@jax.jit
def _embed(idx2d, token_table, pos_table):
    mesh = plsc.VectorSubcoreMesh(core_axis_name="c", subcore_axis_name="s")
    scratch = [
        pltpu.VMEM((CPW, CHUNK), jnp.int32),
        pltpu.VMEM((PATCH_DIM, DIM_MODEL), jnp.float32),
    ] + [pltpu.VMEM((CHUNK, DIM_MODEL), jnp.float32) for _ in range(NBUF)] \
      + [pltpu.SemaphoreType.DMA for _ in range(2 * NBUF)]
    return pl.kernel(
        _body,
        out_type=jax.ShapeDtypeStruct((BATCH * PATCH_DIM, DIM_MODEL),
                                      jnp.float32),
        mesh=mesh,
        scratch_types=scratch,
        compiler_params=pltpu.CompilerParams(use_tc_tiling_on_sc=False),
    )(idx2d, token_table, pos_table)


def kernel(inputs, token_table, pos_table):
    idx2d = inputs.astype(jnp.int32).reshape(N_CHUNKS, CHUNK)
    out = _embed(idx2d, token_table, pos_table)
    return out.reshape(BATCH, PATCH_DIM, DIM_MODEL)


# speedup vs baseline: 4.1998x; 1.3960x over previous
import jax
import jax.numpy as jnp
from jax import lax
from jax.experimental import pallas as pl
from jax.experimental.pallas import tpu as pltpu
from jax.experimental.pallas import tpu_sc as plsc

NUM_PATCHES = 1024
PATCH_DIM = 768
DIM_MODEL = 64
BATCH = 1024

CHUNK = 128
ROWS_PER_WORKER = 64
HALF = PATCH_DIM // 2
CPR = HALF // CHUNK
NCHUNK = ROWS_PER_WORKER * CPR
NBUF = 4
LANES = 16


def _body(idx_hbm, tokp_hbm, pos2_hbm, out_hbm, idx_v, pos_v,
          gbuf0, gbuf1, obuf0, obuf1, gs0, gs1, os0, os1):
    gbufs = (gbuf0, gbuf1)
    obufs = (obuf0, obuf1)
    gsems = (gs0, gs1)
    osems = (os0, os1)

    core = lax.axis_index("c")
    sub = lax.axis_index("s")
    h = core
    b0 = sub * ROWS_PER_WORKER

    pltpu.sync_copy(pos2_hbm.at[pl.ds(h * (HALF // 2), HALF // 2)], pos_v)
    pltpu.sync_copy(idx_hbm.at[pl.ds(b0, ROWS_PER_WORKER), pl.ds(h * HALF, HALF)],
                    idx_v)

    def start_gather(s, i):
        r = lax.div(i, CPR)
        c = lax.rem(i, CPR)
        pltpu.async_copy(tokp_hbm.at[idx_v.at[r, pl.ds(c * CHUNK, CHUNK)]],
                         gbufs[s], gsems[s])

    def wait_gather(s):
        pltpu.make_async_copy(tokp_hbm.at[idx_v.at[0, pl.ds(0, CHUNK)]],
                              gbufs[s], gsems[s]).wait()

    def start_out(s, i):
        r = lax.div(i, CPR)
        c = lax.rem(i, CPR)
        och = (b0 + r) * (2 * CPR) + h * CPR + c
        pltpu.async_copy(obufs[s], out_hbm.at[pl.ds(och * CHUNK, CHUNK)],
                         osems[s])

    def wait_out(s):
        pltpu.make_async_copy(obufs[s], out_hbm.at[pl.ds(0, CHUNK)],
                              osems[s]).wait()

    def add_pos(s, i):
        c = lax.rem(i, CPR)
        qb = c * (CHUNK // 2)
        gbuf = gbufs[s]
        obuf = obufs[s]

        @pl.loop(0, CHUNK // 2, unroll=2)
        def _(rp):
            q = qb + rp
            for par in range(2):
                row = 2 * rp + par
                for cc in range(DIM_MODEL // LANES):
                    x = pos_v[q, pl.ds(par * DIM_MODEL + cc * LANES, LANES)]
                    y = gbuf[row, pl.ds(cc * LANES, LANES)]
                    obuf[row, pl.ds(cc * LANES, LANES)] = x + y

    start_gather(0, 0)
    start_gather(1, 1)
    for i in (0, 1):
        wait_gather(i)
        add_pos(i, i)
        start_out(i, i)
        start_gather(i, i + 2)

    @pl.loop(1, NCHUNK // 2 - 1)
    def _(g):
        for b in range(2):
            i = g * 2 + b
            wait_gather(b)
            wait_out(b)
            add_pos(b, i)
            start_out(b, i)
            start_gather(b, i + 2)

    for b in range(2):
        i = NCHUNK - 2 + b
        wait_gather(b)
        wait_out(b)
        add_pos(b, i)
        start_out(b, i)
    wait_out(0)
    wait_out(1)


@jax.jit
def _embed(idx, tokp, pos2):
    mesh = plsc.VectorSubcoreMesh(core_axis_name="c", subcore_axis_name="s")
    scratch = [
        pltpu.VMEM((ROWS_PER_WORKER, HALF), jnp.int32),
        pltpu.VMEM((HALF // 2, 2 * DIM_MODEL), jnp.float32),
        pltpu.VMEM((CHUNK, 2 * DIM_MODEL), jnp.float32),
        pltpu.VMEM((CHUNK, 2 * DIM_MODEL), jnp.float32),
        pltpu.VMEM((CHUNK, DIM_MODEL), jnp.float32),
        pltpu.VMEM((CHUNK, DIM_MODEL), jnp.float32),
    ] + [pltpu.SemaphoreType.DMA for _ in range(4)]
    return pl.kernel(
        _body,
        out_type=jax.ShapeDtypeStruct((BATCH * PATCH_DIM, DIM_MODEL),
                                      jnp.float32),
        mesh=mesh,
        scratch_types=scratch,
    )(idx, tokp, pos2)


def kernel(inputs, token_table, pos_table):
    idx = inputs.astype(jnp.int32)
    tokp = jnp.pad(token_table, ((0, 0), (0, 2 * DIM_MODEL - DIM_MODEL)))
    pos2 = pos_table.reshape(PATCH_DIM // 2, 2 * DIM_MODEL)
    out = _embed(idx, tokp, pos2)
    return out.reshape(BATCH, PATCH_DIM, DIM_MODEL)


# speedup vs baseline: 15.0607x; 3.5861x over previous
import jax
import jax.numpy as jnp
from jax import lax
from jax.experimental import pallas as pl
from jax.experimental.pallas import tpu as pltpu
from jax.experimental.pallas import tpu_sc as plsc

NUM_PATCHES = 1024
PATCH_DIM = 768
DIM_MODEL = 64
BATCH = 1024

CHUNK = 128
HALF = PATCH_DIM // 2
CPR = HALF // CHUNK
RPW = BATCH // 16
RPB = 8
NBLK = RPW // RPB
LANES = 16
NIV = CHUNK // LANES


def _body(idx_hbm, tokt_hbm, post_hbm, out_hbm, tok_v, pos_v,
          ix0, ix1, ob0, ob1, ob2, is0, is1, os0, os1, os2):
    ixs = (ix0, ix1)
    isems = (is0, is1)
    obufs = (ob0, ob1, ob2)
    osems = (os0, os1, os2)

    h = lax.axis_index("c")
    sub = lax.axis_index("s")
    b0 = sub * RPW
    p0 = h * HALF

    pltpu.sync_copy(tokt_hbm, tok_v)
    pltpu.sync_copy(post_hbm.at[pl.ds(0, DIM_MODEL), pl.ds(p0, HALF)], pos_v)

    def start_idx(slot, blk):
        pltpu.async_copy(
            idx_hbm.at[pl.ds(b0 + blk * RPB, RPB), pl.ds(p0, HALF)],
            ixs[slot], isems[slot])

    def wait_idx(slot):
        pltpu.make_async_copy(idx_hbm.at[pl.ds(0, RPB), pl.ds(0, HALF)],
                              ixs[slot], isems[slot]).wait()

    def start_out(c, b):
        pltpu.async_copy(
            obufs[c],
            out_hbm.at[b, pl.ds(0, DIM_MODEL), pl.ds(p0 + c * CHUNK, CHUNK)],
            osems[c])

    def wait_out(c):
        pltpu.make_async_copy(obufs[c],
                              out_hbm.at[0, pl.ds(0, DIM_MODEL), pl.ds(0, CHUNK)],
                              osems[c]).wait()

    def do_chunk(slot, rr, c, b):
        obuf = obufs[c]
        ivs = [ixs[slot][rr, pl.ds(c * CHUNK + i * LANES, LANES)]
               for i in range(NIV)]

        @plsc.parallel_loop(0, DIM_MODEL, unroll=4)
        def _(d):
            row = jnp.full((LANES,), d, jnp.int32)
            for i in range(NIV):
                g = plsc.load_gather(tok_v, [row, ivs[i]])
                p = pos_v[d, pl.ds(c * CHUNK + i * LANES, LANES)]
                obuf[d, pl.ds(i * LANES, LANES)] = g + p

        start_out(c, b)

    start_idx(0, 0)
    start_idx(1, 1)

    @pl.loop(0, NBLK // 2)
    def _(g):
        for half_blk in range(2):
            blk = g * 2 + half_blk
            slot = half_blk
            wait_idx(slot)

            @pl.loop(0, RPB)
            def _(rr):
                for c in range(CPR):
                    if half_blk == 0:
                        @pl.when(jnp.logical_or(g > 0, rr > 0))
                        def _():
                            wait_out(c)
                    else:
                        wait_out(c)
                    do_chunk(slot, rr, c, b0 + blk * RPB + rr)

            @pl.when(g < NBLK // 2 - 1)
            def _():
                start_idx(slot, blk + 2)

    for c in range(CPR):
        wait_out(c)


@jax.jit
def _embed(idx, tokt, post):
    mesh = plsc.VectorSubcoreMesh(core_axis_name="c", subcore_axis_name="s")
    scratch = [
        pltpu.VMEM((DIM_MODEL, NUM_PATCHES), jnp.float32),
        pltpu.VMEM((DIM_MODEL, HALF), jnp.float32),
        pltpu.VMEM((RPB, HALF), jnp.int32),
        pltpu.VMEM((RPB, HALF), jnp.int32),
        pltpu.VMEM((DIM_MODEL, CHUNK), jnp.float32),
        pltpu.VMEM((DIM_MODEL, CHUNK), jnp.float32),
        pltpu.VMEM((DIM_MODEL, CHUNK), jnp.float32),
    ] + [pltpu.SemaphoreType.DMA for _ in range(5)]
    return pl.kernel(
        _body,
        out_type=jax.ShapeDtypeStruct((BATCH, DIM_MODEL, PATCH_DIM),
                                      jnp.float32),
        mesh=mesh,
        scratch_types=scratch,
        compiler_params=pltpu.CompilerParams(needs_layout_passes=False),
    )(idx, tokt, post)


def kernel(inputs, token_table, pos_table):
    idx = inputs.astype(jnp.int32)
    out = _embed(idx, token_table.T, pos_table.T)
    return out.transpose(0, 2, 1)
